# Initial kernel scaffold; baseline (speedup 1.0000x reference)
import jax
import jax.numpy as jnp
from jax.experimental import pallas as pl


def kernel(param, emb_weight):
    raise NotImplementedError("write your pallas kernel here")


# speedup vs baseline: 5.4358x; 5.4358x over previous
import functools

import jax
import jax.numpy as jnp
from jax import lax
from jax.experimental import pallas as pl
from jax.experimental.pallas import tpu as pltpu
from jax.experimental.pallas import tpu_sc as plsc

ROWS = 16384
COLS = 100
EMB = 16
N = ROWS * COLS
NUM_CORES = 2
NUM_SUBCORES = 16
NW = NUM_CORES * NUM_SUBCORES
PER_W = N // NW
CHUNK = 2048
NCHUNK = PER_W // CHUNK
GROUPS = CHUNK // 16

BINS = (0.0, 0.2, 0.4, 0.6, 0.8, 1.0)
PADDING_IDX = 6

_mesh = plsc.VectorSubcoreMesh(core_axis_name="c", subcore_axis_name="s")


@functools.partial(
    pl.kernel,
    mesh=_mesh,
    out_type=jax.ShapeDtypeStruct((N * EMB,), jnp.float32),
    scratch_types=[
        pltpu.VMEM((7, EMB), jnp.float32),
        pltpu.VMEM((CHUNK,), jnp.float32),
        pltpu.VMEM((CHUNK * EMB,), jnp.float32),
    ],
    compiler_params=pltpu.CompilerParams(needs_layout_passes=False),
)
def _sc_embed(param_hbm, emb_hbm, out_hbm, emb_v, in_v, out_v):
    wid = lax.axis_index("s") * NUM_CORES + lax.axis_index("c")
    base = wid * PER_W
    pltpu.sync_copy(emb_hbm, emb_v)

    iota = lax.iota(jnp.int32, 16)
    row_stride = iota * jnp.full((16,), EMB, jnp.int32)
    ones = jnp.full((16,), 1, jnp.int32)
    zeros = jnp.full((16,), 0, jnp.int32)
    pad_vec = jnp.full((16,), PADDING_IDX, jnp.int32)
    bin_vecs = [jnp.full((16,), b, jnp.float32) for b in BINS]

    def chunk_body(c, carry):
        off = base + c * CHUNK
        pltpu.sync_copy(param_hbm.at[pl.ds(off, CHUNK)], in_v)

        def group_body(g, carry2):
            v = in_v[pl.ds(g * 16, 16)]
            idx = zeros
            for bv in bin_vecs:
                idx = idx + jnp.where(v > bv, ones, zeros)
            idx = jnp.where(v != v, pad_vec, idx)
            gbase = g * (16 * EMB)
            for k in range(EMB):
                col = plsc.load_gather(
                    emb_v, [idx, jnp.full((16,), k, jnp.int32)])
                plsc.store_scatter(
                    out_v,
                    [row_stride + jnp.full((16,), gbase + k, jnp.int32)],
                    col)
            return carry2

        lax.fori_loop(0, GROUPS, group_body, 0)
        pltpu.sync_copy(out_v, out_hbm.at[pl.ds(off * EMB, CHUNK * EMB)])
        return carry

    lax.fori_loop(0, NCHUNK, chunk_body, 0)


def kernel(param, emb_weight):
    out = _sc_embed(param.reshape(-1), emb_weight)
    return out.reshape(ROWS, COLS, EMB)
